# baseline (speedup 1.0000x reference)
import jax
import jax.numpy as jnp
from jax.experimental import pallas as pl
from jax.experimental.pallas import tpu as pltpu

TOP_K = 8
ROW_BLOCK = 8192


def _gating_body(x_ref, w_ref, b_ref, gatest_ref, idxt_ref):
    logits = (
        jnp.dot(x_ref[...], w_ref[...], preferred_element_type=jnp.float32)
        + b_ref[...]
    )
    lt = logits.T
    e = lt.shape[0]
    iota = jax.lax.broadcasted_iota(jnp.int32, lt.shape, 0)
    work = lt
    mask = jnp.zeros(lt.shape, jnp.bool_)
    idx_rows = []
    for _ in range(TOP_K):
        m = jnp.max(work, axis=0, keepdims=True)
        sel_idx = jnp.min(
            jnp.where(work == m, iota, e), axis=0, keepdims=True
        )
        idx_rows.append(sel_idx)
        sel = iota == sel_idx
        mask = mask | sel
        work = jnp.where(sel, -jnp.inf, work)
    top1 = jnp.max(lt, axis=0, keepdims=True)
    ex = jnp.where(mask, jnp.exp(lt - top1), 0.0)
    gatest_ref[...] = ex / jnp.sum(ex, axis=0, keepdims=True)
    idxt_ref[...] = jnp.concatenate(idx_rows, axis=0)


def kernel(x, W, b):
    n_tok, d = x.shape
    e = W.shape[1]
    b2 = b.reshape(1, e)
    grid = (n_tok // ROW_BLOCK,)
    gates_t, idx_t = pl.pallas_call(
        _gating_body,
        grid=grid,
        in_specs=[
            pl.BlockSpec((ROW_BLOCK, d), lambda i: (i, 0)),
            pl.BlockSpec((d, e), lambda i: (0, 0)),
            pl.BlockSpec((1, e), lambda i: (0, 0)),
        ],
        out_specs=[
            pl.BlockSpec((e, ROW_BLOCK), lambda i: (0, i)),
            pl.BlockSpec((TOP_K, ROW_BLOCK), lambda i: (0, i)),
        ],
        out_shape=[
            jax.ShapeDtypeStruct((e, n_tok), jnp.float32),
            jax.ShapeDtypeStruct((TOP_K, n_tok), jnp.int32),
        ],
        compiler_params=pltpu.CompilerParams(
            dimension_semantics=("parallel",),
        ),
    )(x, W, b2)
    return (gates_t.T, idx_t.T)


# speedup vs baseline: 1.0851x; 1.0851x over previous
import jax
import jax.numpy as jnp
from jax.experimental import pallas as pl
from jax.experimental.pallas import tpu as pltpu

TOP_K = 8
ROW_BLOCK = 4096


def _gating_body(x_ref, w_ref, b_ref, gatest_ref, idxt_ref):
    logits = (
        jnp.dot(x_ref[...], w_ref[...], preferred_element_type=jnp.float32)
        + b_ref[...]
    )
    lt = logits.T
    e = lt.shape[0]
    iota = jax.lax.broadcasted_iota(jnp.int32, lt.shape, 0)
    work = lt
    top1 = None
    idx_rows = []
    for _ in range(TOP_K):
        m = jnp.max(work, axis=0, keepdims=True)
        if top1 is None:
            top1 = m
        sel_idx = jnp.min(
            jnp.where(work == m, iota, e), axis=0, keepdims=True
        )
        idx_rows.append(sel_idx)
        work = jnp.where(iota == sel_idx, -jnp.inf, work)
    ex = jnp.where(jnp.isneginf(work), jnp.exp(lt - top1), 0.0)
    gatest_ref[...] = ex / jnp.sum(ex, axis=0, keepdims=True)
    idxt_ref[...] = jnp.concatenate(idx_rows, axis=0)


def kernel(x, W, b):
    n_tok, d = x.shape
    e = W.shape[1]
    b2 = b.reshape(1, e)
    grid = (n_tok // ROW_BLOCK,)
    gates_t, idx_t = pl.pallas_call(
        _gating_body,
        grid=grid,
        in_specs=[
            pl.BlockSpec((ROW_BLOCK, d), lambda i: (i, 0)),
            pl.BlockSpec((d, e), lambda i: (0, 0)),
            pl.BlockSpec((1, e), lambda i: (0, 0)),
        ],
        out_specs=[
            pl.BlockSpec((e, ROW_BLOCK), lambda i: (0, i)),
            pl.BlockSpec((TOP_K, ROW_BLOCK), lambda i: (0, i)),
        ],
        out_shape=[
            jax.ShapeDtypeStruct((e, n_tok), jnp.float32),
            jax.ShapeDtypeStruct((TOP_K, n_tok), jnp.int32),
        ],
        compiler_params=pltpu.CompilerParams(
            dimension_semantics=("parallel",),
        ),
    )(x, W, b2)
    return (gates_t.T, idx_t.T)
